# baseline (speedup 1.0000x reference)
import jax
import jax.numpy as jnp
from jax.experimental import pallas as pl
from jax.experimental.pallas import tpu as pltpu


def _make_gcn_kernel(N, S, F, nA, rows_c, bm):
    cols = S * F

    def _gcn_kernel(adj_ref, x_ref, w_ref, b_ref, o_ref, xb_ref):
        i = pl.program_id(0)

        @pl.when(i < nA)
        def _prep():
            xw = jnp.dot(x_ref[...].astype(jnp.bfloat16),
                         w_ref[...].astype(jnp.bfloat16),
                         preferred_element_type=jnp.float32)
            xb_ref[pl.ds(i * rows_c, rows_c), :] = (
                xw.astype(jnp.bfloat16).reshape(rows_c, cols))

        @pl.when(i >= nA)
        def _main():
            a = adj_ref[...].astype(jnp.bfloat16)
            t = jnp.dot(a, xb_ref[...], preferred_element_type=jnp.float32)
            o_ref[...] = t.reshape(bm, S, F) + b_ref[...]

    return _gcn_kernel


def kernel(x, adj, weight, bias):
    N, S, F = x.shape
    cols = S * F

    x2d = x.reshape(N * S, F)
    b_row = bias.reshape(1, 1, F).astype(jnp.float32)

    bm = 512 if N % 512 == 0 else N
    nB = N // bm
    nA = 8 if (N % 8 == 0 and (N // 8) % 8 == 0) else 1
    rows_c = N // nA

    return pl.pallas_call(
        _make_gcn_kernel(N, S, F, nA, rows_c, bm),
        out_shape=jax.ShapeDtypeStruct((N, S, F), x.dtype),
        grid=(nA + nB,),
        in_specs=[
            pl.BlockSpec((bm, N),
                         lambda i: (jnp.maximum(i - nA, 0), 0)),
            pl.BlockSpec((rows_c * S, F),
                         lambda i: (jnp.minimum(i, nA - 1), 0)),
            pl.BlockSpec((F, F), lambda i: (0, 0)),
            pl.BlockSpec((1, 1, F), lambda i: (0, 0, 0)),
        ],
        out_specs=pl.BlockSpec(
            (bm, S, F), lambda i: (jnp.maximum(i - nA, 0), 0, 0)),
        scratch_shapes=[pltpu.VMEM((N, cols), jnp.bfloat16)],
        compiler_params=pltpu.CompilerParams(
            dimension_semantics=("arbitrary",)),
    )(adj, x2d, weight, b_row)


# speedup vs baseline: 1.1127x; 1.1127x over previous
import jax
import jax.numpy as jnp
from jax.experimental import pallas as pl
from jax.experimental.pallas import tpu as pltpu


def _make_gcn_kernel(N, S, F, nA, rows_c, bm):
    cols = S * F

    def _gcn_kernel(adj_ref, x_ref, w_ref, b_ref, o_ref, xb_ref):
        i = pl.program_id(0)

        @pl.when(i < nA)
        def _prep():
            xw = jnp.dot(x_ref[...].astype(jnp.bfloat16),
                         w_ref[...].astype(jnp.bfloat16),
                         preferred_element_type=jnp.float32)
            xb_ref[pl.ds(i * rows_c, rows_c), :] = (
                xw.astype(jnp.bfloat16).reshape(rows_c, cols))

        @pl.when(i >= nA)
        def _main():
            a = adj_ref[...].astype(jnp.bfloat16)
            t = jnp.dot(a, xb_ref[...], preferred_element_type=jnp.float32)
            o_ref[...] = t.reshape(bm, S, F) + b_ref[...]

    return _gcn_kernel


def kernel(x, adj, weight, bias):
    N, S, F = x.shape
    cols = S * F

    x2d = x.reshape(N * S, F)
    b_row = bias.reshape(1, 1, F).astype(jnp.float32)

    bm = 512 if N % 512 == 0 else N
    nB = N // bm
    nA = 4 if (N % 4 == 0 and (N // 4) % 8 == 0) else 1
    rows_c = N // nA

    return pl.pallas_call(
        _make_gcn_kernel(N, S, F, nA, rows_c, bm),
        out_shape=jax.ShapeDtypeStruct((N, S, F), x.dtype),
        grid=(nA + nB,),
        in_specs=[
            pl.BlockSpec((bm, N),
                         lambda i: (jnp.maximum(i - nA, 0), 0)),
            pl.BlockSpec((rows_c * S, F),
                         lambda i: (jnp.minimum(i, nA - 1), 0)),
            pl.BlockSpec((F, F), lambda i: (0, 0)),
            pl.BlockSpec((1, 1, F), lambda i: (0, 0, 0)),
        ],
        out_specs=pl.BlockSpec(
            (bm, S, F), lambda i: (jnp.maximum(i - nA, 0), 0, 0)),
        scratch_shapes=[pltpu.VMEM((N, cols), jnp.bfloat16)],
        compiler_params=pltpu.CompilerParams(
            dimension_semantics=("arbitrary",)),
    )(adj, x2d, weight, b_row)
